# baseline (speedup 1.0000x reference)
import functools
import jax
import jax.numpy as jnp
from jax import lax
from jax.experimental import pallas as pl
from jax.experimental.pallas import tpu as pltpu
from jax.experimental.pallas import tpu_sc as plsc

B, S, EMB = 4096, 200, 64
NC, NS = 2, 16
NW = NC * NS
ROWS_PER_W = B // NW
CB = 2
CHUNK = CB * S
NCHUNK = ROWS_PER_W // CB
LANES = 16
COLS = EMB // LANES


def _sc_call(idx, word_table, pos2):
    mesh = plsc.VectorSubcoreMesh(core_axis_name="c", subcore_axis_name="s")

    @functools.partial(
        pl.kernel,
        mesh=mesh,
        compiler_params=pltpu.CompilerParams(use_tc_tiling_on_sc=False),
        out_type=jax.ShapeDtypeStruct((B * S, EMB), jnp.float32),
        scratch_types=[
            pltpu.VMEM((CHUNK,), jnp.int32),
            pltpu.VMEM((CHUNK, EMB), jnp.float32),
            pltpu.VMEM((CHUNK, EMB), jnp.float32),
            pltpu.SemaphoreType.DMA,
        ],
    )
    def k(idx_hbm, table_hbm, pos_hbm, out_hbm, idx_v, rows_v, pos_v, sem):
        wid = lax.axis_index("s") * NC + lax.axis_index("c")
        base = wid * ROWS_PER_W * S
        pltpu.sync_copy(pos_hbm, pos_v)

        def body(g, carry):
            off = base + g * CHUNK
            pltpu.sync_copy(idx_hbm.at[pl.ds(off, CHUNK)], idx_v)
            pltpu.async_copy(table_hbm.at[idx_v], rows_v, sem).wait()

            def add_body(r, c2):
                for c in range(COLS):
                    sl = pl.ds(c * LANES, LANES)
                    rows_v[r, sl] = rows_v[r, sl] + pos_v[r, sl]
                return c2

            lax.fori_loop(0, CHUNK, add_body, 0, unroll=2)
            pltpu.sync_copy(rows_v, out_hbm.at[pl.ds(off, CHUNK)])
            return carry

        lax.fori_loop(0, NCHUNK, body, 0)

    return k(idx, word_table, pos2)


def kernel(X, word_table, position_embedding):
    idx = X.reshape(-1).astype(jnp.int32)
    pos2 = jnp.tile(position_embedding[:S, :], (CB, 1))
    out = _sc_call(idx, word_table, pos2)
    return out.reshape(B, S, EMB)


# speedup vs baseline: 1.2565x; 1.2565x over previous
import functools
import jax
import jax.numpy as jnp
from jax import lax
from jax.experimental import pallas as pl
from jax.experimental.pallas import tpu as pltpu
from jax.experimental.pallas import tpu_sc as plsc

B, S, EMB = 4096, 200, 64
NC, NS = 2, 16
NW = NC * NS
ROWS_PER_W = B // NW
CB = 2
CHUNK = CB * S
NCHUNK = ROWS_PER_W // CB
NBUF = 2
NT = NCHUNK // NBUF
W_IDX = ROWS_PER_W * S
LANES = 16
COLS = EMB // LANES


def _sc_call(idx, word_table, pos):
    mesh = plsc.VectorSubcoreMesh(core_axis_name="c", subcore_axis_name="s")

    @functools.partial(
        pl.kernel,
        mesh=mesh,
        compiler_params=pltpu.CompilerParams(use_tc_tiling_on_sc=False),
        out_type=jax.ShapeDtypeStruct((B * S, EMB), jnp.float32),
        scratch_types=[
            pltpu.VMEM((W_IDX,), jnp.int32),
            pltpu.VMEM((CHUNK, EMB), jnp.float32),
            pltpu.VMEM((CHUNK, EMB), jnp.float32),
            pltpu.VMEM((S, EMB), jnp.float32),
            pltpu.SemaphoreType.DMA,
            pltpu.SemaphoreType.DMA,
            pltpu.SemaphoreType.DMA,
            pltpu.SemaphoreType.DMA,
        ],
    )
    def k(idx_hbm, table_hbm, pos_hbm, out_hbm,
          idx_v, rows0, rows1, pos_v, g0, g1, w0, w1):
        rows = [rows0, rows1]
        gs = [g0, g1]
        ws = [w0, w1]
        wid = lax.axis_index("s") * NC + lax.axis_index("c")
        base = wid * W_IDX
        pltpu.sync_copy(idx_hbm.at[pl.ds(base, W_IDX)], idx_v)
        pltpu.sync_copy(pos_hbm, pos_v)

        def idx_slice(g):
            return idx_v.at[pl.ds(g * CHUNK, CHUNK)]

        def out_slice(g):
            return out_hbm.at[pl.ds(base + g * CHUNK, CHUNK)]

        def start_gather(g, b):
            pltpu.async_copy(table_hbm.at[idx_slice(g)], rows[b], gs[b])

        def wait_gather(g, b):
            pltpu.make_async_copy(table_hbm.at[idx_slice(g)], rows[b], gs[b]).wait()

        def start_wb(g, b):
            pltpu.async_copy(rows[b], out_slice(g), ws[b])

        def wait_wb(g, b):
            pltpu.make_async_copy(rows[b], out_slice(g), ws[b]).wait()

        def add_pos(b):
            rbuf = rows[b]

            def body(r, c):
                for c4 in range(COLS):
                    sl = pl.ds(c4 * LANES, LANES)
                    p = pos_v[r, sl]
                    for cb in range(CB):
                        rr = r + cb * S
                        rbuf[rr, sl] = rbuf[rr, sl] + p
                return c

            lax.fori_loop(0, S, body, 0, unroll=4)

        for b in range(NBUF):
            start_gather(b, b)

        def body(t, carry):
            for b in range(NBUF):
                g = t * NBUF + b
                wait_gather(g, b)
                add_pos(b)
                start_wb(g, b)
                wait_wb(g, b)
                start_gather(g + NBUF, b)
            return carry

        lax.fori_loop(0, NT - 1, body, 0)

        for b in range(NBUF):
            g = (NT - 1) * NBUF + b
            wait_gather(g, b)
            add_pos(b)
            start_wb(g, b)
        for b in range(NBUF):
            wait_wb((NT - 1) * NBUF + b, b)

    return k(idx, word_table, pos)


def kernel(X, word_table, position_embedding):
    idx = X.reshape(-1).astype(jnp.int32)
    pos = position_embedding[:S, :]
    out = _sc_call(idx, word_table, pos)
    return out.reshape(B, S, EMB)
